# baseline (speedup 1.0000x reference)
import jax
import jax.numpy as jnp
from jax import lax
from jax.experimental import pallas as pl
from jax.experimental.pallas import tpu as pltpu
from jax.experimental.pallas import tpu_sc as plsc

NUM_CLASSES_ = 100000
FEATURE_DIM_ = 64
BATCH_ = 16384

NC = 2
NS = 16
L = 16

EPT = BATCH_ // NS
CH = 128
NCHUNK = EPT // CH
W = FEATURE_DIM_ + L
NP = 3
PS = 16768
RPT = PS // NS
HALF = NP * PS
ACC_T = NC * HALF
ACC_ROWS = PS + 1


def _sc_kernel(inputs_hbm, labels_hbm, centers_hbm,
               acc_hbm, loss_hbm,
               lbl_v, cb_v, inp_v, db_v, idx_v, zb_v, lp_v, acc_sh, sem):
    c = lax.axis_index("c")
    s = lax.axis_index("s")
    base_e = pl.multiple_of(s * EPT, CH)

    ones16 = jnp.full((L,), 1.0, jnp.float32)
    zeros16 = jnp.zeros((L,), jnp.float32)

    pltpu.sync_copy(labels_hbm.at[pl.ds(base_e, EPT)], lbl_v)

    def _zb(r, _):
        for q in range(W // L):
            zb_v[r, pl.ds(q * L, L)] = zeros16
        return 0
    lax.fori_loop(0, CH, _zb, 0)

    accs = (zeros16, zeros16, zeros16, zeros16)
    lo = pl.multiple_of(s * RPT, 8)
    for p in range(NP):
        base_cls = c * HALF + p * PS

        nfull, rem = divmod(RPT, CH)
        for k in range(nfull):
            pltpu.sync_copy(zb_v, acc_sh.at[pl.ds(lo + k * CH, CH)])
        if rem:
            pltpu.sync_copy(zb_v.at[pl.ds(0, rem), :],
                            acc_sh.at[pl.ds(lo + nfull * CH, rem)])

        def _idx(k, _):
            v = lbl_v[pl.ds(k * L, L)]
            rel = v - base_cls
            ok = (rel >= 0) & (rel < PS)
            idx_v[k // (CH // L), pl.ds((k % (CH // L)) * L, L)] = (
                jnp.where(ok, rel, PS))
            return 0
        lax.fori_loop(0, EPT // L, _idx, 0)

        plsc.subcore_barrier()

        for j in range(NCHUNK):
            pltpu.async_copy(
                centers_hbm.at[lbl_v.at[pl.ds(j * CH, CH)]], cb_v, sem).wait()
            pltpu.sync_copy(
                inputs_hbm.at[pl.ds(base_e + j * CH, CH), :], inp_v)

            def _row(r, acc):
                out = []
                for q in range(FEATURE_DIM_ // L):
                    cv = cb_v[r, pl.ds(q * L, L)]
                    iv = inp_v[r, pl.ds(q * L, L)]
                    d = cv - iv
                    db_v[r, pl.ds(q * L, L)] = d
                    out.append(acc[q] + d * d if p == 0 else acc[q])
                db_v[r, pl.ds(FEATURE_DIM_, L)] = ones16
                return tuple(out)
            accs = lax.fori_loop(0, CH, _row, accs)

            pltpu.sync_copy(db_v, acc_sh.at[idx_v.at[j]], add=True)

        plsc.subcore_barrier()

        pltpu.sync_copy(acc_sh.at[pl.ds(lo, RPT)],
                        acc_hbm.at[pl.ds(pl.multiple_of(base_cls + lo, 8),
                                         RPT)])

        plsc.subcore_barrier()

    lp_v[0, :] = accs[0] + accs[1] + accs[2] + accs[3]
    wid = c * NS + s
    pltpu.sync_copy(lp_v, loss_hbm.at[wid])


def _tc_body(cen_ref, acc_ref, lp_ref, out_ref, loss_ref):
    acc = acc_ref[...]
    out_ref[...] = cen_ref[...] - acc[:, :FEATURE_DIM_] / (
        acc[:, FEATURE_DIM_:FEATURE_DIM_ + 1] + 1.0)

    @pl.when(pl.program_id(0) == 0)
    def _():
        loss_ref[...] = jnp.reshape(
            jnp.sum(lp_ref[...]) / (2.0 * BATCH_), (1, 1))


def kernel(inputs, labels, centers):
    labels = jnp.reshape(labels, (-1,)).astype(jnp.int32)

    mesh = plsc.VectorSubcoreMesh(core_axis_name="c", subcore_axis_name="s")
    sc = pl.kernel(
        _sc_kernel,
        out_type=(
            jax.ShapeDtypeStruct((ACC_T, W), jnp.float32),
            jax.ShapeDtypeStruct((NC * NS, 1, L), jnp.float32),
        ),
        mesh=mesh,
        compiler_params=pltpu.CompilerParams(use_tc_tiling_on_sc=False),
        scratch_types=[
            pltpu.VMEM((EPT,), jnp.int32),
            pltpu.VMEM((CH, FEATURE_DIM_), jnp.float32),
            pltpu.VMEM((CH, FEATURE_DIM_), jnp.float32),
            pltpu.VMEM((CH, W), jnp.float32),
            pltpu.VMEM((NCHUNK, CH), jnp.int32),
            pltpu.VMEM((CH, W), jnp.float32),
            pltpu.VMEM((1, L), jnp.float32),
            pltpu.VMEM_SHARED((ACC_ROWS, W), jnp.float32),
            pltpu.SemaphoreType.DMA,
        ],
    )
    acc, loss_part = sc(inputs, labels, centers)
    loss_part = jnp.reshape(loss_part, (NC * NS, L))

    rows = 2000
    new_centers, loss = pl.pallas_call(
        _tc_body,
        grid=(NUM_CLASSES_ // rows,),
        in_specs=[
            pl.BlockSpec((rows, FEATURE_DIM_), lambda i: (i, 0)),
            pl.BlockSpec((rows, W), lambda i: (i, 0)),
            pl.BlockSpec((NC * NS, L), lambda i: (0, 0)),
        ],
        out_specs=[
            pl.BlockSpec((rows, FEATURE_DIM_), lambda i: (i, 0)),
            pl.BlockSpec((1, 1), lambda i: (0, 0)),
        ],
        out_shape=[
            jax.ShapeDtypeStruct((NUM_CLASSES_, FEATURE_DIM_), jnp.float32),
            jax.ShapeDtypeStruct((1, 1), jnp.float32),
        ],
    )(centers, acc, loss_part)

    return inputs, new_centers, jnp.reshape(loss, ())


# speedup vs baseline: 1.4870x; 1.4870x over previous
import jax
import jax.numpy as jnp
from jax import lax
from jax.experimental import pallas as pl
from jax.experimental.pallas import tpu as pltpu
from jax.experimental.pallas import tpu_sc as plsc

NUM_CLASSES_ = 100000
FEATURE_DIM_ = 64
BATCH_ = 16384

NC = 2
NS = 16
L = 16

EPT = BATCH_ // NS
CH = 128
NCHUNK = EPT // CH
W = FEATURE_DIM_ + L
NP = 3
PS = 16768
RPT = PS // NS
HALF = NP * PS
ACC_T = NC * HALF
ACC_ROWS = PS + 1
NBLK = 50
RB = NUM_CLASSES_ // NBLK


def _sc_kernel(inputs_hbm, labels_hbm,
               acc_hbm, loss_hbm,
               lbl_v, sb_v, idx_v, zb_v, lp_v, acc_sh, sem):
    c = lax.axis_index("c")
    s = lax.axis_index("s")
    base_e = pl.multiple_of(s * EPT, CH)

    ones16 = jnp.full((L,), 1.0, jnp.float32)
    zeros16 = jnp.zeros((L,), jnp.float32)

    pltpu.sync_copy(labels_hbm.at[pl.ds(base_e, EPT)], lbl_v)

    def _zb(r, _):
        for q in range(W // L):
            zb_v[r, pl.ds(q * L, L)] = zeros16
        sb_v[r, pl.ds(FEATURE_DIM_, L)] = ones16
        return 0
    lax.fori_loop(0, CH, _zb, 0)

    accs = (zeros16, zeros16, zeros16, zeros16)
    lo = pl.multiple_of(s * RPT, 8)
    for p in range(NP):
        base_cls = c * HALF + p * PS

        nfull, rem = divmod(RPT, CH)
        for k in range(nfull):
            pltpu.sync_copy(zb_v, acc_sh.at[pl.ds(lo + k * CH, CH)])
        if rem:
            pltpu.sync_copy(zb_v.at[pl.ds(0, rem), :],
                            acc_sh.at[pl.ds(lo + nfull * CH, rem)])

        def _idx(k, _):
            v = lbl_v[pl.ds(k * L, L)]
            rel = v - base_cls
            ok = (rel >= 0) & (rel < PS)
            idx_v[k // (CH // L), pl.ds((k % (CH // L)) * L, L)] = (
                jnp.where(ok, rel, PS))
            return 0
        lax.fori_loop(0, EPT // L, _idx, 0)

        plsc.subcore_barrier()

        for j in range(NCHUNK):
            pltpu.sync_copy(
                inputs_hbm.at[pl.ds(base_e + j * CH, CH), :],
                sb_v.at[:, pl.ds(0, FEATURE_DIM_)])
            if p == 0:
                def _row(r, acc):
                    out = []
                    for q in range(FEATURE_DIM_ // L):
                        v = sb_v[r, pl.ds(q * L, L)]
                        out.append(acc[q] + v * v)
                    return tuple(out)
                accs = lax.fori_loop(0, CH, _row, accs)
            pltpu.sync_copy(sb_v, acc_sh.at[idx_v.at[j]], add=True)

        plsc.subcore_barrier()

        pltpu.sync_copy(acc_sh.at[pl.ds(lo, RPT)],
                        acc_hbm.at[pl.ds(pl.multiple_of(base_cls + lo, 8),
                                         RPT)])

        plsc.subcore_barrier()

    lp_v[0, :] = accs[0] + accs[1] + accs[2] + accs[3]
    wid = c * NS + s
    pltpu.sync_copy(lp_v, loss_hbm.at[wid])


def _tc_body(cen_ref, acc_ref, lp_ref, out_ref, loss_ref):
    acc = acc_ref[...]
    xsum = acc[:, :FEATURE_DIM_]
    cnt = acc[:, FEATURE_DIM_:FEATURE_DIM_ + 1]
    cen = cen_ref[...]
    num = cnt * cen - xsum
    out_ref[...] = cen - num / (cnt + 1.0)

    part = jnp.sum((cnt * cen - 2.0 * xsum) * cen)

    i = pl.program_id(0)

    @pl.when(i == 0)
    def _():
        loss_ref[...] = jnp.reshape(jnp.sum(lp_ref[...]) * 0.5, (1, 1))

    loss_ref[...] = loss_ref[...] + jnp.reshape(part, (1, 1))

    @pl.when(i == NBLK - 1)
    def _():
        loss_ref[...] = loss_ref[...] * (1.0 / BATCH_)


def kernel(inputs, labels, centers):
    labels = jnp.reshape(labels, (-1,)).astype(jnp.int32)

    mesh = plsc.VectorSubcoreMesh(core_axis_name="c", subcore_axis_name="s")
    sc = pl.kernel(
        _sc_kernel,
        out_type=(
            jax.ShapeDtypeStruct((ACC_T, W), jnp.float32),
            jax.ShapeDtypeStruct((NC * NS, 1, L), jnp.float32),
        ),
        mesh=mesh,
        compiler_params=pltpu.CompilerParams(use_tc_tiling_on_sc=False),
        scratch_types=[
            pltpu.VMEM((EPT,), jnp.int32),
            pltpu.VMEM((CH, W), jnp.float32),
            pltpu.VMEM((NCHUNK, CH), jnp.int32),
            pltpu.VMEM((CH, W), jnp.float32),
            pltpu.VMEM((1, L), jnp.float32),
            pltpu.VMEM_SHARED((ACC_ROWS, W), jnp.float32),
            pltpu.SemaphoreType.DMA,
        ],
    )
    acc, loss_part = sc(inputs, labels)
    loss_part = jnp.reshape(loss_part, (NC * NS, L))

    new_centers, loss = pl.pallas_call(
        _tc_body,
        grid=(NBLK,),
        in_specs=[
            pl.BlockSpec((RB, FEATURE_DIM_), lambda i: (i, 0)),
            pl.BlockSpec((RB, W), lambda i: (i, 0)),
            pl.BlockSpec((NC * NS, L), lambda i: (0, 0)),
        ],
        out_specs=[
            pl.BlockSpec((RB, FEATURE_DIM_), lambda i: (i, 0)),
            pl.BlockSpec((1, 1), lambda i: (0, 0)),
        ],
        out_shape=[
            jax.ShapeDtypeStruct((NUM_CLASSES_, FEATURE_DIM_), jnp.float32),
            jax.ShapeDtypeStruct((1, 1), jnp.float32),
        ],
    )(centers, acc, loss_part)

    return inputs, new_centers, jnp.reshape(loss, ())


# speedup vs baseline: 1.7712x; 1.1912x over previous
import jax
import jax.numpy as jnp
from jax import lax
from jax.experimental import pallas as pl
from jax.experimental.pallas import tpu as pltpu
from jax.experimental.pallas import tpu_sc as plsc

NUM_CLASSES_ = 100000
FEATURE_DIM_ = 64
BATCH_ = 16384

NC = 2
NS = 16
L = 16

EPT = BATCH_ // NS
CH = 128
NCHUNK = EPT // CH
W = FEATURE_DIM_ + L
NP = 3
PS = 16768
RPT = PS // NS
HALF = NP * PS
ACC_T = NC * HALF
ACC_ROWS = PS + 1
NBLK = 50
RB = NUM_CLASSES_ // NBLK


def _sc_kernel(inputs_hbm, labels_hbm,
               acc_hbm, loss_hbm,
               lbl_v, sb_v, idx_v, zb_v, lp_v, acc_sh, sem):
    c = lax.axis_index("c")
    s = lax.axis_index("s")
    base_e = pl.multiple_of(s * EPT, CH)

    ones16 = jnp.full((L,), 1.0, jnp.float32)
    zeros16 = jnp.zeros((L,), jnp.float32)

    pltpu.sync_copy(labels_hbm.at[pl.ds(base_e, EPT)], lbl_v)

    def _zb(r, _):
        for q in range(W // L):
            zb_v[r, pl.ds(q * L, L)] = zeros16
        sb_v[r, pl.ds(FEATURE_DIM_, L)] = ones16
        return 0
    lax.fori_loop(0, CH, _zb, 0)

    accs = (zeros16, zeros16, zeros16, zeros16)
    lo = pl.multiple_of(s * RPT, 8)
    for p in range(NP):
        base_cls = c * HALF + p * PS

        nfull, rem = divmod(RPT, CH)
        for k in range(nfull):
            pltpu.sync_copy(zb_v, acc_sh.at[pl.ds(lo + k * CH, CH)])
        if rem:
            pltpu.sync_copy(zb_v.at[pl.ds(0, rem), :],
                            acc_sh.at[pl.ds(lo + nfull * CH, rem)])

        def _idx(k, _):
            v = lbl_v[pl.ds(k * L, L)]
            rel = v - base_cls
            ok = (rel >= 0) & (rel < PS)
            idx_v[k // (CH // L), pl.ds((k % (CH // L)) * L, L)] = (
                jnp.where(ok, rel, PS))
            return 0
        lax.fori_loop(0, EPT // L, _idx, 0)

        plsc.subcore_barrier()

        for j in range(NCHUNK):
            pltpu.sync_copy(
                inputs_hbm.at[pl.ds(base_e + j * CH, CH), :],
                sb_v.at[:, pl.ds(0, FEATURE_DIM_)])
            if p == 0:
                def _row(r, acc):
                    out = []
                    for q in range(FEATURE_DIM_ // L):
                        v = sb_v[r, pl.ds(q * L, L)]
                        out.append(acc[q] + v * v)
                    return tuple(out)
                accs = lax.fori_loop(0, CH, _row, accs)
            pltpu.sync_copy(sb_v, acc_sh.at[idx_v.at[j]], add=True)

        plsc.subcore_barrier()

        pltpu.sync_copy(acc_sh.at[pl.ds(lo, RPT)],
                        acc_hbm.at[pl.ds(pl.multiple_of(base_cls + lo, 8),
                                         RPT), pl.ds(0, W)])

        plsc.subcore_barrier()

    lp_v[0, :] = accs[0] + accs[1] + accs[2] + accs[3]
    wid = c * NS + s
    pltpu.sync_copy(lp_v, loss_hbm.at[wid])


def _tc_body(cen_ref, acc_ref, lp_ref, out_ref, loss_ref):
    acc = acc_ref[...]
    xsum = acc[:, :FEATURE_DIM_]
    cnt = acc[:, FEATURE_DIM_:FEATURE_DIM_ + 1]
    cen = cen_ref[...]
    num = cnt * cen - xsum
    out_ref[...] = cen - num / (cnt + 1.0)

    part = jnp.sum((cnt * cen - 2.0 * xsum) * cen)

    i = pl.program_id(0)

    @pl.when(i == 0)
    def _():
        loss_ref[...] = jnp.reshape(jnp.sum(lp_ref[...]) * 0.5, (1, 1))

    loss_ref[...] = loss_ref[...] + jnp.reshape(part, (1, 1))

    @pl.when(i == NBLK - 1)
    def _():
        loss_ref[...] = loss_ref[...] * (1.0 / BATCH_)


def kernel(inputs, labels, centers):
    labels = jnp.reshape(labels, (-1,)).astype(jnp.int32)

    mesh = plsc.VectorSubcoreMesh(core_axis_name="c", subcore_axis_name="s")
    sc = pl.kernel(
        _sc_kernel,
        out_type=(
            jax.ShapeDtypeStruct((ACC_T, 128), jnp.float32),
            jax.ShapeDtypeStruct((NC * NS, 1, L), jnp.float32),
        ),
        mesh=mesh,
        compiler_params=pltpu.CompilerParams(use_tc_tiling_on_sc=False),
        scratch_types=[
            pltpu.VMEM((EPT,), jnp.int32),
            pltpu.VMEM((CH, W), jnp.float32),
            pltpu.VMEM((NCHUNK, CH), jnp.int32),
            pltpu.VMEM((CH, W), jnp.float32),
            pltpu.VMEM((1, L), jnp.float32),
            pltpu.VMEM_SHARED((ACC_ROWS, W), jnp.float32),
            pltpu.SemaphoreType.DMA,
        ],
    )
    acc, loss_part = sc(inputs, labels)
    loss_part = jnp.reshape(loss_part, (NC * NS, L))

    new_centers, loss = pl.pallas_call(
        _tc_body,
        grid=(NBLK,),
        in_specs=[
            pl.BlockSpec((RB, FEATURE_DIM_), lambda i: (i, 0)),
            pl.BlockSpec((RB, 128), lambda i: (i, 0)),
            pl.BlockSpec((NC * NS, L), lambda i: (0, 0)),
        ],
        out_specs=[
            pl.BlockSpec((RB, FEATURE_DIM_), lambda i: (i, 0)),
            pl.BlockSpec((1, 1), lambda i: (0, 0)),
        ],
        out_shape=[
            jax.ShapeDtypeStruct((NUM_CLASSES_, FEATURE_DIM_), jnp.float32),
            jax.ShapeDtypeStruct((1, 1), jnp.float32),
        ],
    )(centers, acc, loss_part)

    return inputs, new_centers, jnp.reshape(loss, ())
